# Initial kernel scaffold; baseline (speedup 1.0000x reference)
import jax
import jax.numpy as jnp
from jax.experimental import pallas as pl


def kernel(pre_embedding, preembed_mask, embed_table, W, b):
    raise NotImplementedError("write your pallas kernel here")


# speedup vs baseline: 4.7065x; 4.7065x over previous
import functools

import numpy as np
import jax
import jax.numpy as jnp
from jax import lax
from jax.experimental import pallas as pl
from jax.experimental.pallas import tpu as pltpu
from jax.experimental.pallas import tpu_sc as plsc


def _sinusoidal_pe(seq_len, d_model):
    pos = np.arange(seq_len, dtype=np.float32)[:, None]
    div = np.exp(np.arange(0, d_model, 2, dtype=np.float32) * (-np.log(10000.0) / d_model))
    pe = np.zeros((seq_len, d_model), dtype=np.float32)
    pe[:, 0::2] = np.sin(pos * div)
    pe[:, 1::2] = np.cos(pos * div)
    return pe


def _mm_body(x_ref, w_ref, o_ref):
    o_ref[...] = jnp.dot(x_ref[...], w_ref[...], preferred_element_type=jnp.float32)


def _table_transform(table, w_top):
    v, e = table.shape
    bm = 2000
    assert v % bm == 0
    return pl.pallas_call(
        _mm_body,
        grid=(v // bm,),
        in_specs=[
            pl.BlockSpec((bm, e), lambda i: (i, 0)),
            pl.BlockSpec((e, e), lambda i: (0, 0)),
        ],
        out_specs=pl.BlockSpec((bm, e), lambda i: (i, 0)),
        out_shape=jax.ShapeDtypeStruct((v, e), jnp.float32),
    )(table, w_top)


def _pos_body(pe_ref, w_ref, b_ref, o_ref):
    o_ref[...] = (
        jnp.dot(pe_ref[...], w_ref[...], preferred_element_type=jnp.float32)
        + b_ref[...]
    )


def _pos_transform(pe, w_bot, b):
    s, e = pe.shape
    return pl.pallas_call(
        _pos_body,
        in_specs=[
            pl.BlockSpec((s, e), lambda: (0, 0)),
            pl.BlockSpec((e, e), lambda: (0, 0)),
            pl.BlockSpec((1, e), lambda: (0, 0)),
        ],
        out_specs=pl.BlockSpec((s, e), lambda: (0, 0)),
        out_shape=jax.ShapeDtypeStruct((s, e), jnp.float32),
    )(pe, w_bot, b.reshape(1, e))


def _sc_gather(idx, table2, pos2, B, S, E):
    info = plsc.get_sparse_core_info()
    NC, NS = info.num_cores, info.num_subcores
    NW = NC * NS
    assert B % NW == 0
    bpw = B // NW

    n0 = 128
    n1 = S - n0

    mesh = plsc.VectorSubcoreMesh(core_axis_name="c", subcore_axis_name="s")

    @functools.partial(
        pl.kernel,
        mesh=mesh,
        out_type=jax.ShapeDtypeStruct((B * S, E), jnp.float32),
        scratch_types=[
            pltpu.VMEM((S,), jnp.int32),
            pltpu.VMEM((S, E), jnp.float32),
            pltpu.VMEM((S, E), jnp.float32),
            pltpu.SemaphoreType.DMA,
        ],
    )
    def k(idx_hbm, table2_hbm, pos2_hbm, out_hbm, idx_v, rows_v, pos_v, sem):
        wid = lax.axis_index("s") * NC + lax.axis_index("c")
        pltpu.sync_copy(pos2_hbm, pos_v)

        def batch_body(i, carry):
            bb = wid * bpw + i
            pltpu.sync_copy(idx_hbm.at[bb], idx_v)
            cp0 = pltpu.async_copy(
                table2_hbm.at[idx_v.at[pl.ds(0, n0)]], rows_v.at[pl.ds(0, n0)], sem
            )
            cp1 = pltpu.async_copy(
                table2_hbm.at[idx_v.at[pl.ds(n0, n1)]], rows_v.at[pl.ds(n0, n1)], sem
            )
            cp0.wait()
            cp1.wait()

            def add_body(r, c2):
                for c in range(E // 16):
                    plsc.addupdate(
                        rows_v.at[r, pl.ds(c * 16, 16)], pos_v[r, pl.ds(c * 16, 16)]
                    )
                return c2

            lax.fori_loop(0, S, add_body, 0)
            pltpu.sync_copy(rows_v, out_hbm.at[pl.ds(bb * S, S)])
            return carry

        lax.fori_loop(0, bpw, batch_body, 0)

    return k(idx, table2, pos2)


def kernel(pre_embedding, preembed_mask, embed_table, W, b):
    B, S = pre_embedding.shape
    V, E = embed_table.shape
    w_top = W[:E, :]
    w_bot = W[E:, :]
    pe = jnp.asarray(_sinusoidal_pe(S, E))

    table2 = _table_transform(embed_table, w_top)
    pos2 = _pos_transform(pe, w_bot, b)
    idx = pre_embedding.astype(jnp.int32)
    out = _sc_gather(idx, table2, pos2, B, S, E)
    return (out.reshape(B, S, E), preembed_mask)


# speedup vs baseline: 7.8357x; 1.6649x over previous
import functools

import numpy as np
import jax
import jax.numpy as jnp
from jax import lax
from jax.experimental import pallas as pl
from jax.experimental.pallas import tpu as pltpu
from jax.experimental.pallas import tpu_sc as plsc


def _sinusoidal_pe(seq_len, d_model):
    pos = np.arange(seq_len, dtype=np.float32)[:, None]
    div = np.exp(np.arange(0, d_model, 2, dtype=np.float32) * (-np.log(10000.0) / d_model))
    pe = np.zeros((seq_len, d_model), dtype=np.float32)
    pe[:, 0::2] = np.sin(pos * div)
    pe[:, 1::2] = np.cos(pos * div)
    return pe


def _mm_body(x_ref, w_ref, o_ref):
    o_ref[...] = jnp.dot(x_ref[...], w_ref[...], preferred_element_type=jnp.float32)


def _table_transform(table, w_top):
    v, e = table.shape
    bm = 2000
    assert v % bm == 0
    return pl.pallas_call(
        _mm_body,
        grid=(v // bm,),
        in_specs=[
            pl.BlockSpec((bm, e), lambda i: (i, 0)),
            pl.BlockSpec((e, e), lambda i: (0, 0)),
        ],
        out_specs=pl.BlockSpec((bm, e), lambda i: (i, 0)),
        out_shape=jax.ShapeDtypeStruct((v, e), jnp.float32),
    )(table, w_top)


def _pos_body(pe_ref, w_ref, b_ref, o_ref):
    o_ref[...] = (
        jnp.dot(pe_ref[...], w_ref[...], preferred_element_type=jnp.float32)
        + b_ref[...]
    )


def _pos_transform(pe, w_bot, b):
    s, e = pe.shape
    return pl.pallas_call(
        _pos_body,
        in_specs=[
            pl.BlockSpec((s, e), lambda: (0, 0)),
            pl.BlockSpec((e, e), lambda: (0, 0)),
            pl.BlockSpec((1, e), lambda: (0, 0)),
        ],
        out_specs=pl.BlockSpec((s, e), lambda: (0, 0)),
        out_shape=jax.ShapeDtypeStruct((s, e), jnp.float32),
    )(pe, w_bot, b.reshape(1, e))


def _sc_gather(idx, table2, pos2, B, S, E):
    info = plsc.get_sparse_core_info()
    NC, NS = info.num_cores, info.num_subcores
    NW = NC * NS
    assert B % NW == 0
    bpw = B // NW

    n0 = 128
    n1 = S - n0

    mesh = plsc.VectorSubcoreMesh(core_axis_name="c", subcore_axis_name="s")

    @functools.partial(
        pl.kernel,
        mesh=mesh,
        out_type=jax.ShapeDtypeStruct((B * S, E), jnp.float32),
        scratch_types=[
            pltpu.VMEM((bpw * S,), jnp.int32),
            pltpu.VMEM((S, E), jnp.float32),
            pltpu.VMEM((S, E), jnp.float32),
            pltpu.VMEM((S, E), jnp.float32),
            pltpu.SemaphoreType.DMA,
            pltpu.SemaphoreType.DMA,
        ],
    )
    def k(idx_hbm, table2_hbm, pos2_hbm, out_hbm, idx_v, pos_v, buf0, buf1, sem_g, sem_w):
        wid = lax.axis_index("s") * NC + lax.axis_index("c")
        base_b = wid * bpw
        pltpu.sync_copy(pos2_hbm, pos_v)
        pltpu.sync_copy(idx_hbm.at[pl.ds(base_b * S, bpw * S)], idx_v)
        bufs = (buf0, buf1)

        def gather_descs(i, buf):
            off = i * S
            return (
                pltpu.make_async_copy(
                    table2_hbm.at[idx_v.at[pl.ds(off, n0)]], buf.at[pl.ds(0, n0)], sem_g
                ),
                pltpu.make_async_copy(
                    table2_hbm.at[idx_v.at[pl.ds(off + n0, n1)]],
                    buf.at[pl.ds(n0, n1)],
                    sem_g,
                ),
            )

        def out_desc(i, buf):
            return pltpu.make_async_copy(
                buf, out_hbm.at[pl.ds((base_b + i) * S, S)], sem_w
            )

        def add_pos(buf):
            def rbody(r4, c2):
                for dr in range(4):
                    r = r4 * 4 + dr
                    for c in range(E // 16):
                        plsc.addupdate(
                            buf.at[r, pl.ds(c * 16, 16)], pos_v[r, pl.ds(c * 16, 16)]
                        )
                return c2

            lax.fori_loop(0, S // 4, rbody, 0)

        for d in gather_descs(0, buf0):
            d.start()

        def body(j, carry):
            for h in range(2):
                i = 2 * j + h
                buf = bufs[h]
                nbuf = bufs[1 - h]
                for d in gather_descs(i, buf):
                    d.wait()

                @pl.when(i + 1 < bpw)
                def _():
                    @pl.when(i >= 1)
                    def __():
                        out_desc(i - 1, nbuf).wait()

                    for d in gather_descs(i + 1, nbuf):
                        d.start()

                add_pos(buf)
                out_desc(i, buf).start()
            return carry

        lax.fori_loop(0, bpw // 2, body, 0)
        out_desc(bpw - 2, buf0).wait()
        out_desc(bpw - 1, buf1).wait()

    return k(idx.reshape(B * S), table2, pos2)


def kernel(pre_embedding, preembed_mask, embed_table, W, b):
    B, S = pre_embedding.shape
    V, E = embed_table.shape
    w_top = W[:E, :]
    w_bot = W[E:, :]
    pe = jnp.asarray(_sinusoidal_pe(S, E))

    table2 = _table_transform(embed_table, w_top)
    pos2 = _pos_transform(pe, w_bot, b)
    idx = pre_embedding.astype(jnp.int32)
    out = _sc_gather(idx, table2, pos2, B, S, E)
    return (out.reshape(B, S, E), preembed_mask)
